# baseline (speedup 1.0000x reference)
import functools

import jax
import jax.numpy as jnp
from jax import lax
from jax.experimental import pallas as pl
from jax.experimental.pallas import tpu as pltpu
from jax.experimental.pallas import tpu_sc as plsc

HIDDEN = 128
LANES = 16
NJ = HIDDEN // LANES
EPS = 1e-5
NUM_CORES = 2
NUM_SUBCORES = 16
NUM_WORKERS = NUM_CORES * NUM_SUBCORES
CHUNK = 128


@functools.lru_cache(maxsize=None)
def _build(n_tokens: int, vocab: int, small_rows: int, pos_rows: int,
           seg_rows: int, n_chunks_ceil: int):
  n_per_w = n_tokens // NUM_WORKERS
  n_chunks = n_per_w // CHUNK
  assert n_per_w % CHUNK == 0 and n_tokens % NUM_WORKERS == 0
  seg_base = pos_rows * HIDDEN
  age_base = (pos_rows + seg_rows) * HIDDEN

  mesh = plsc.VectorSubcoreMesh(
      core_axis_name="c", subcore_axis_name="s",
      num_cores=NUM_CORES, num_subcores=NUM_SUBCORES)

  @functools.partial(
      pl.kernel,
      mesh=mesh,
      compiler_params=pltpu.CompilerParams(needs_layout_passes=False),
      out_type=jax.ShapeDtypeStruct((n_tokens, HIDDEN), jnp.float32),
      scratch_types=[
          pltpu.VMEM((CHUNK,), jnp.int32),
          pltpu.VMEM((CHUNK,), jnp.int32),
          pltpu.VMEM((CHUNK,), jnp.int32),
          pltpu.VMEM((CHUNK,), jnp.int32),
          pltpu.VMEM((CHUNK, HIDDEN), jnp.float32),
          pltpu.VMEM((small_rows * HIDDEN,), jnp.float32),
          pltpu.VMEM((2 * HIDDEN,), jnp.float32),
          pltpu.SemaphoreType.DMA,
      ],
  )
  def k(iw_hbm, ip_hbm, is_hbm, ia_hbm, word_hbm, small_hbm, gb_hbm,
        out_hbm, iw_v, ip_v, is_v, ia_v, rows_v, small_v, gb_v, sem):
    wid = lax.axis_index("s") * NUM_CORES + lax.axis_index("c")
    base_w = wid * n_per_w

    pltpu.sync_copy(small_hbm, small_v)
    pltpu.sync_copy(gb_hbm, gb_v)

    iota = lax.iota(jnp.int32, 16)
    gammas = [gb_v[pl.ds(j * LANES, LANES)] for j in range(NJ)]
    betas = [gb_v[pl.ds(HIDDEN + j * LANES, LANES)] for j in range(NJ)]

    def chunk_body(ci, carry):
      base = base_w + ci * CHUNK
      pltpu.sync_copy(iw_hbm.at[pl.ds(base, CHUNK)], iw_v)
      pltpu.sync_copy(ip_hbm.at[pl.ds(base, CHUNK)], ip_v)
      pltpu.sync_copy(is_hbm.at[pl.ds(base, CHUNK)], is_v)
      pltpu.sync_copy(ia_hbm.at[pl.ds(base, CHUNK)], ia_v)
      pltpu.async_copy(word_hbm.at[iw_v], rows_v, sem).wait()

      def group_body(g, gcarry):
        r0 = g * LANES
        rows16 = r0 + iota
        pb_v = ip_v[pl.ds(r0, LANES)] * HIDDEN
        sb_v = seg_base + is_v[pl.ds(r0, LANES)] * HIDDEN
        ab_v = age_base + ia_v[pl.ds(r0, LANES)] * HIDDEN

        def col_body(c, accs):
          acc_s, acc_q = accs
          c16 = jnp.full((LANES,), c, jnp.int32)
          wc = plsc.load_gather(rows_v, [rows16, c16])
          pc = plsc.load_gather(small_v, [pb_v + c])
          sc = plsc.load_gather(small_v, [sb_v + c])
          ac = plsc.load_gather(small_v, [ab_v + c])
          xc = wc + pc + sc + ac
          plsc.store_scatter(rows_v, [rows16, c16], xc)
          return acc_s + xc, acc_q + xc * xc

        zero = jnp.zeros((LANES,), jnp.float32)
        acc_s, acc_q = lax.fori_loop(0, HIDDEN, col_body, (zero, zero),
                                     unroll=4)
        mean16 = acc_s * (1.0 / HIDDEN)
        var16 = acc_q * (1.0 / HIDDEN) - mean16 * mean16
        v_v = var16 + EPS
        yi = jnp.int32(0x5F3759DF) - (plsc.bitcast(v_v, jnp.int32) >> 1)
        y = plsc.bitcast(yi, jnp.float32)
        half_v = v_v * 0.5
        for _ in range(3):
          y = y * (1.5 - half_v * y * y)
        for t in range(LANES):
          r = r0 + t
          mean_v = jnp.full((LANES,), mean16[t], jnp.float32)
          rstd_v = jnp.full((LANES,), y[t], jnp.float32)
          for j in range(NJ):
            x = rows_v[r, pl.ds(j * LANES, LANES)]
            xh = (x - mean_v) * rstd_v
            rows_v[r, pl.ds(j * LANES, LANES)] = xh * gammas[j] + betas[j]
        return gcarry

      lax.fori_loop(0, CHUNK // LANES, group_body, 0)
      pltpu.sync_copy(rows_v, out_hbm.at[pl.ds(base, CHUNK)])
      return carry

    lax.fori_loop(0, n_chunks, chunk_body, 0)

  return k


def kernel(input_ids, position_ids, segment_ids, age_ids, word_table,
           pos_table, seg_table, age_table, ln_gamma, ln_beta):
  b, s = input_ids.shape
  n_tokens = b * s
  iw = input_ids.reshape(-1).astype(jnp.int32)
  ip = position_ids.reshape(-1).astype(jnp.int32)
  iseg = segment_ids.reshape(-1).astype(jnp.int32)
  ia = age_ids.reshape(-1).astype(jnp.int32)
  small = jnp.concatenate(
      [pos_table, seg_table, age_table], axis=0).reshape(-1)
  gb = jnp.concatenate([ln_gamma, ln_beta], axis=0)
  pos_rows = pos_table.shape[0]
  seg_rows = seg_table.shape[0]
  small_rows = pos_rows + seg_rows + age_table.shape[0]
  k = _build(n_tokens, word_table.shape[0], small_rows, pos_rows, seg_rows, 0)
  out = k(iw, ip, iseg, ia, word_table, small, gb)
  return out.reshape(b, s, HIDDEN)


# speedup vs baseline: 4.4899x; 4.4899x over previous
import functools

import jax
import jax.numpy as jnp
from jax import lax
from jax.experimental import pallas as pl
from jax.experimental.pallas import tpu as pltpu
from jax.experimental.pallas import tpu_sc as plsc

HIDDEN = 128
LANES = 16
NJ = HIDDEN // LANES
EPS = 1e-5
NUM_CORES = 2
NUM_SUBCORES = 16
NUM_WORKERS = NUM_CORES * NUM_SUBCORES
CHUNK = 128


@functools.lru_cache(maxsize=None)
def _build(n_tokens: int, vocab: int, small_rows: int, pos_rows: int,
           seg_rows: int, n_chunks_ceil: int):
  n_per_w = n_tokens // NUM_WORKERS
  n_chunks = n_per_w // CHUNK
  assert n_per_w % CHUNK == 0 and n_tokens % NUM_WORKERS == 0
  seg_base = pos_rows * HIDDEN
  age_base = (pos_rows + seg_rows) * HIDDEN

  mesh = plsc.VectorSubcoreMesh(
      core_axis_name="c", subcore_axis_name="s",
      num_cores=NUM_CORES, num_subcores=NUM_SUBCORES)

  @functools.partial(
      pl.kernel,
      mesh=mesh,
      compiler_params=pltpu.CompilerParams(needs_layout_passes=False),
      out_type=jax.ShapeDtypeStruct((n_tokens, HIDDEN), jnp.float32),
      scratch_types=[
          pltpu.VMEM((CHUNK,), jnp.int32),
          pltpu.VMEM((CHUNK,), jnp.int32),
          pltpu.VMEM((CHUNK,), jnp.int32),
          pltpu.VMEM((CHUNK,), jnp.int32),
          pltpu.VMEM((CHUNK, HIDDEN), jnp.float32),
          pltpu.VMEM((small_rows * HIDDEN,), jnp.float32),
          pltpu.VMEM((2 * HIDDEN,), jnp.float32),
          pltpu.SemaphoreType.DMA,
      ],
  )
  def k(iw_hbm, ip_hbm, is_hbm, ia_hbm, word_hbm, small_hbm, gb_hbm,
        out_hbm, iw_v, ip_v, is_v, ia_v, rows_v, small_v, gb_v, sem):
    wid = lax.axis_index("s") * NUM_CORES + lax.axis_index("c")
    base_w = wid * n_per_w

    pltpu.sync_copy(small_hbm, small_v)
    pltpu.sync_copy(gb_hbm, gb_v)

    iota = lax.iota(jnp.int32, 16)
    perms = [jnp.bitwise_xor(iota, jnp.int32(1 << kk)) for kk in range(4)]
    gammas = [gb_v[pl.ds(j * LANES, LANES)] for j in range(NJ)]
    betas = [gb_v[pl.ds(HIDDEN + j * LANES, LANES)] for j in range(NJ)]

    def chunk_body(ci, carry):
      base = base_w + ci * CHUNK
      pltpu.sync_copy(iw_hbm.at[pl.ds(base, CHUNK)], iw_v)
      pltpu.sync_copy(ip_hbm.at[pl.ds(base, CHUNK)], ip_v)
      pltpu.sync_copy(is_hbm.at[pl.ds(base, CHUNK)], is_v)
      pltpu.sync_copy(ia_hbm.at[pl.ds(base, CHUNK)], ia_v)
      pltpu.async_copy(word_hbm.at[iw_v], rows_v, sem).wait()

      def group_body(g, gcarry):
        r0 = g * LANES
        pb_v = ip_v[pl.ds(r0, LANES)] * HIDDEN
        sb_v = seg_base + is_v[pl.ds(r0, LANES)] * HIDDEN
        ab_v = age_base + ia_v[pl.ds(r0, LANES)] * HIDDEN
        for t in range(LANES):
          r = r0 + t
          pbase = pb_v[t]
          sbase = sb_v[t]
          abase = ab_v[t]
          xs = []
          sum_v = None
          sumsq_v = None
          for j in range(NJ):
            off = j * LANES
            w = rows_v[r, pl.ds(off, LANES)]
            p = plsc.load_gather(small_v, [pbase + off + iota])
            s = plsc.load_gather(small_v, [sbase + off + iota])
            a = plsc.load_gather(small_v, [abase + off + iota])
            x = w + p + s + a
            xs.append(x)
            sum_v = x if sum_v is None else sum_v + x
            sumsq_v = x * x if sumsq_v is None else sumsq_v + x * x
          for perm in perms:
            sum_v = sum_v + sum_v.at[perm].get(mode="promise_in_bounds")
            sumsq_v = sumsq_v + sumsq_v.at[perm].get(
                mode="promise_in_bounds")
          mean_v = sum_v * (1.0 / HIDDEN)
          var_v = sumsq_v * (1.0 / HIDDEN) - mean_v * mean_v
          v_v = var_v + EPS
          yi = jnp.int32(0x5F3759DF) - (plsc.bitcast(v_v, jnp.int32) >> 1)
          y = plsc.bitcast(yi, jnp.float32)
          half_v = v_v * 0.5
          for _ in range(3):
            y = y * (1.5 - half_v * y * y)
          for j in range(NJ):
            xh = (xs[j] - mean_v) * y
            rows_v[r, pl.ds(j * LANES, LANES)] = xh * gammas[j] + betas[j]
        return gcarry

      lax.fori_loop(0, CHUNK // LANES, group_body, 0)
      pltpu.sync_copy(rows_v, out_hbm.at[pl.ds(base, CHUNK)])
      return carry

    lax.fori_loop(0, n_chunks, chunk_body, 0)

  return k


def kernel(input_ids, position_ids, segment_ids, age_ids, word_table,
           pos_table, seg_table, age_table, ln_gamma, ln_beta):
  b, s = input_ids.shape
  n_tokens = b * s
  iw = input_ids.reshape(-1).astype(jnp.int32)
  ip = position_ids.reshape(-1).astype(jnp.int32)
  iseg = segment_ids.reshape(-1).astype(jnp.int32)
  ia = age_ids.reshape(-1).astype(jnp.int32)
  small = jnp.concatenate(
      [pos_table, seg_table, age_table], axis=0).reshape(-1)
  gb = jnp.concatenate([ln_gamma, ln_beta], axis=0)
  pos_rows = pos_table.shape[0]
  seg_rows = seg_table.shape[0]
  small_rows = pos_rows + seg_rows + age_table.shape[0]
  k = _build(n_tokens, word_table.shape[0], small_rows, pos_rows, seg_rows, 0)
  out = k(iw, ip, iseg, ia, word_table, small, gb)
  return out.reshape(b, s, HIDDEN)


# speedup vs baseline: 5.1071x; 1.1375x over previous
import functools

import jax
import jax.numpy as jnp
from jax import lax
from jax.experimental import pallas as pl
from jax.experimental.pallas import tpu as pltpu
from jax.experimental.pallas import tpu_sc as plsc

HIDDEN = 128
LANES = 16
NJ = HIDDEN // LANES
EPS = 1e-5
NUM_CORES = 2
NUM_SUBCORES = 16
NUM_WORKERS = NUM_CORES * NUM_SUBCORES
CHUNK = 128


@functools.lru_cache(maxsize=None)
def _build(n_tokens: int, small_rows: int, pos_rows: int, seg_rows: int):
  n_per_w = n_tokens // NUM_WORKERS
  n_chunks = n_per_w // CHUNK
  assert n_tokens % NUM_WORKERS == 0 and n_per_w % CHUNK == 0
  assert n_chunks % 2 == 0
  seg_base = pos_rows * HIDDEN
  age_base = (pos_rows + seg_rows) * HIDDEN

  mesh = plsc.VectorSubcoreMesh(
      core_axis_name="c", subcore_axis_name="s",
      num_cores=NUM_CORES, num_subcores=NUM_SUBCORES)

  @functools.partial(
      pl.kernel,
      mesh=mesh,
      compiler_params=pltpu.CompilerParams(needs_layout_passes=False),
      out_type=jax.ShapeDtypeStruct((n_tokens, HIDDEN), jnp.float32),
      scratch_types=[
          pltpu.VMEM((4, CHUNK), jnp.int32),
          pltpu.VMEM((4, CHUNK), jnp.int32),
          pltpu.VMEM((CHUNK, HIDDEN), jnp.float32),
          pltpu.VMEM((CHUNK, HIDDEN), jnp.float32),
          pltpu.VMEM((small_rows * HIDDEN,), jnp.float32),
          pltpu.VMEM((2 * HIDDEN,), jnp.float32),
          pltpu.SemaphoreType.DMA,
          pltpu.SemaphoreType.DMA,
          pltpu.SemaphoreType.DMA,
          pltpu.SemaphoreType.DMA,
          pltpu.SemaphoreType.DMA,
          pltpu.SemaphoreType.DMA,
      ],
  )
  def k(idx4_hbm, word_hbm, small_hbm, gb_hbm, out_hbm,
        idx0_v, idx1_v, rows0_v, rows1_v, small_v, gb_v,
        isem0, isem1, gsem0, gsem1, wsem0, wsem1):
    wid = lax.axis_index("s") * NUM_CORES + lax.axis_index("c")
    base_w = wid * n_per_w
    blk_w = wid * n_chunks

    pltpu.sync_copy(small_hbm, small_v)
    pltpu.sync_copy(gb_hbm, gb_v)

    iota = lax.iota(jnp.int32, 16)
    perms = [jnp.bitwise_xor(iota, jnp.int32(1 << kk)) for kk in range(4)]
    gammas = [gb_v[pl.ds(j * LANES, LANES)] for j in range(NJ)]
    betas = [gb_v[pl.ds(HIDDEN + j * LANES, LANES)] for j in range(NJ)]

    def compute(idx_v, rows_v):
      def group_body(g, gcarry):
        r0 = g * LANES
        pb_v = idx_v[1, pl.ds(r0, LANES)] * HIDDEN
        sb_v = seg_base + idx_v[2, pl.ds(r0, LANES)] * HIDDEN
        ab_v = age_base + idx_v[3, pl.ds(r0, LANES)] * HIDDEN
        for t in range(LANES):
          r = r0 + t
          pbase = pb_v[t]
          sbase = sb_v[t]
          abase = ab_v[t]
          xs = []
          sum_v = None
          sumsq_v = None
          for j in range(NJ):
            off = j * LANES
            w = rows_v[r, pl.ds(off, LANES)]
            p = plsc.load_gather(small_v, [pbase + off + iota])
            s = plsc.load_gather(small_v, [sbase + off + iota])
            a = plsc.load_gather(small_v, [abase + off + iota])
            x = w + p + s + a
            xs.append(x)
            sum_v = x if sum_v is None else sum_v + x
            sumsq_v = x * x if sumsq_v is None else sumsq_v + x * x
          for perm in perms:
            sum_v = sum_v + sum_v.at[perm].get(mode="promise_in_bounds")
            sumsq_v = sumsq_v + sumsq_v.at[perm].get(
                mode="promise_in_bounds")
          mean_v = sum_v * (1.0 / HIDDEN)
          var_v = sumsq_v * (1.0 / HIDDEN) - mean_v * mean_v
          v_v = var_v + EPS
          yi = jnp.int32(0x5F3759DF) - (plsc.bitcast(v_v, jnp.int32) >> 1)
          y = plsc.bitcast(yi, jnp.float32)
          half_v = v_v * 0.5
          for _ in range(3):
            y = y * (1.5 - half_v * y * y)
          for j in range(NJ):
            xh = (xs[j] - mean_v) * y
            rows_v[r, pl.ds(j * LANES, LANES)] = xh * gammas[j] + betas[j]
        return gcarry

      lax.fori_loop(0, CHUNK // LANES, group_body, 0)

    def phase(ci, idx_cur, idx_nxt, rows_cur, rows_nxt,
              isem_nxt, gsem_cur, gsem_nxt, wsem_cur, wsem_nxt):
      base = base_w + ci * CHUNK

      @pl.when(ci < n_chunks - 1)
      def _():
        pltpu.async_copy(idx4_hbm.at[blk_w + ci + 1], idx_nxt, isem_nxt)

      @pl.when(ci > 0)
      def _():
        pltpu.make_async_copy(
            rows_nxt, out_hbm.at[pl.ds(base, CHUNK)], wsem_nxt).wait()

      @pl.when(ci < n_chunks - 1)
      def _():
        pltpu.make_async_copy(
            idx4_hbm.at[blk_w + ci + 1], idx_nxt, isem_nxt).wait()
        pltpu.async_copy(word_hbm.at[idx_nxt.at[0]], rows_nxt, gsem_nxt)

      pltpu.make_async_copy(
          word_hbm.at[idx_cur.at[0]], rows_cur, gsem_cur).wait()
      compute(idx_cur, rows_cur)
      pltpu.async_copy(rows_cur, out_hbm.at[pl.ds(base, CHUNK)], wsem_cur)

    pltpu.sync_copy(idx4_hbm.at[blk_w], idx0_v)
    pltpu.async_copy(word_hbm.at[idx0_v.at[0]], rows0_v, gsem0)

    def loop_body(i, c):
      ci = i * 2
      phase(ci, idx0_v, idx1_v, rows0_v, rows1_v,
            isem1, gsem0, gsem1, wsem0, wsem1)
      phase(ci + 1, idx1_v, idx0_v, rows1_v, rows0_v,
            isem0, gsem1, gsem0, wsem1, wsem0)
      return c

    lax.fori_loop(0, n_chunks // 2, loop_body, 0)
    last_base = base_w + (n_chunks - 1) * CHUNK
    pltpu.make_async_copy(
        rows1_v, out_hbm.at[pl.ds(last_base, CHUNK)], wsem1).wait()

  return k


def kernel(input_ids, position_ids, segment_ids, age_ids, word_table,
           pos_table, seg_table, age_table, ln_gamma, ln_beta):
  b, s = input_ids.shape
  n_tokens = b * s
  n_blocks = n_tokens // CHUNK
  idx4 = jnp.stack([
      input_ids.reshape(n_blocks, CHUNK),
      position_ids.reshape(n_blocks, CHUNK),
      segment_ids.reshape(n_blocks, CHUNK),
      age_ids.reshape(n_blocks, CHUNK),
  ], axis=1).astype(jnp.int32)
  small = jnp.concatenate(
      [pos_table, seg_table, age_table], axis=0).reshape(-1)
  gb = jnp.concatenate([ln_gamma, ln_beta], axis=0)
  pos_rows = pos_table.shape[0]
  seg_rows = seg_table.shape[0]
  small_rows = pos_rows + seg_rows + age_table.shape[0]
  k = _build(n_tokens, small_rows, pos_rows, seg_rows)
  out = k(idx4, word_table, small, gb)
  return out.reshape(b, s, HIDDEN)


# speedup vs baseline: 6.0167x; 1.1781x over previous
import functools

import jax
import jax.numpy as jnp
from jax import lax
from jax.experimental import pallas as pl
from jax.experimental.pallas import tpu as pltpu
from jax.experimental.pallas import tpu_sc as plsc

HIDDEN = 128
LANES = 16
NJ = HIDDEN // LANES
EPS = 1e-5
NUM_CORES = 2
NUM_SUBCORES = 16
NUM_WORKERS = NUM_CORES * NUM_SUBCORES
CHUNK = 128


@functools.lru_cache(maxsize=None)
def _build(n_tokens: int, small_rows: int, pos_rows: int, seg_rows: int):
  n_per_w = n_tokens // NUM_WORKERS
  n_chunks = n_per_w // CHUNK
  assert n_tokens % NUM_WORKERS == 0 and n_per_w % CHUNK == 0
  assert n_chunks % 2 == 0
  seg_base = pos_rows * HIDDEN
  age_base = (pos_rows + seg_rows) * HIDDEN

  mesh = plsc.VectorSubcoreMesh(
      core_axis_name="c", subcore_axis_name="s",
      num_cores=NUM_CORES, num_subcores=NUM_SUBCORES)

  @functools.partial(
      pl.kernel,
      mesh=mesh,
      compiler_params=pltpu.CompilerParams(needs_layout_passes=False),
      out_type=jax.ShapeDtypeStruct((n_tokens, HIDDEN), jnp.float32),
      scratch_types=[
          pltpu.VMEM((4, CHUNK), jnp.int32),
          pltpu.VMEM((4, CHUNK), jnp.int32),
          pltpu.VMEM((CHUNK, HIDDEN), jnp.float32),
          pltpu.VMEM((CHUNK, HIDDEN), jnp.float32),
          pltpu.VMEM((small_rows * HIDDEN,), jnp.float32),
          pltpu.VMEM((2 * HIDDEN,), jnp.float32),
          pltpu.SemaphoreType.DMA,
          pltpu.SemaphoreType.DMA,
          pltpu.SemaphoreType.DMA,
          pltpu.SemaphoreType.DMA,
          pltpu.SemaphoreType.DMA,
          pltpu.SemaphoreType.DMA,
      ],
  )
  def k(idx4_hbm, word_hbm, small_hbm, gb_hbm, out_hbm,
        idx0_v, idx1_v, rows0_v, rows1_v, small_v, gb_v,
        isem0, isem1, gsem0, gsem1, wsem0, wsem1):
    wid = lax.axis_index("s") * NUM_CORES + lax.axis_index("c")
    base_w = wid * n_per_w
    blk_w = wid * n_chunks

    pltpu.sync_copy(small_hbm, small_v)
    pltpu.sync_copy(gb_hbm, gb_v)

    iota = lax.iota(jnp.int32, 16)
    perms = [jnp.bitwise_xor(iota, jnp.int32(1 << kk)) for kk in range(4)]
    gammas = [gb_v[pl.ds(j * LANES, LANES)] for j in range(NJ)]
    betas = [gb_v[pl.ds(HIDDEN + j * LANES, LANES)] for j in range(NJ)]
    seg0 = [small_v[pl.ds(seg_base + j * LANES, LANES)] for j in range(NJ)]
    seg1 = [small_v[pl.ds(seg_base + HIDDEN + j * LANES, LANES)]
            for j in range(NJ)]

    def compute(idx_v, rows_v):
      @plsc.parallel_loop(0, CHUNK // LANES)
      def _(g):
        r0 = g * LANES
        pb_v = idx_v[1, pl.ds(r0, LANES)] * HIDDEN
        sid_v = idx_v[2, pl.ds(r0, LANES)]
        ab_v = age_base + idx_v[3, pl.ds(r0, LANES)] * HIDDEN
        for t in range(LANES):
          r = r0 + t
          pbase = pb_v[t]
          use_seg1 = sid_v[t] != 0
          abase = ab_v[t]
          xs = []
          sum_v = None
          sumsq_v = None
          for j in range(NJ):
            off = j * LANES
            w = rows_v[r, pl.ds(off, LANES)]
            p = plsc.load_gather(small_v, [pbase + off + iota])
            a = plsc.load_gather(small_v, [abase + off + iota])
            s = jnp.where(use_seg1, seg1[j], seg0[j])
            x = (w + p) + (s + a)
            xs.append(x)
            sum_v = x if sum_v is None else sum_v + x
            sumsq_v = x * x if sumsq_v is None else sumsq_v + x * x
          for perm in perms:
            sum_v = sum_v + sum_v.at[perm].get(mode="promise_in_bounds")
            sumsq_v = sumsq_v + sumsq_v.at[perm].get(
                mode="promise_in_bounds")
          mean_v = sum_v * (1.0 / HIDDEN)
          var_v = sumsq_v * (1.0 / HIDDEN) - mean_v * mean_v
          v_v = var_v + EPS
          yi = jnp.int32(0x5F3759DF) - (plsc.bitcast(v_v, jnp.int32) >> 1)
          y = plsc.bitcast(yi, jnp.float32)
          half_v = v_v * 0.5
          for _ in range(2):
            y = y * (1.5 - half_v * y * y)
          for j in range(NJ):
            xh = (xs[j] - mean_v) * y
            rows_v[r, pl.ds(j * LANES, LANES)] = xh * gammas[j] + betas[j]

    def phase(ci, idx_cur, idx_nxt, rows_cur, rows_nxt,
              isem_nxt, gsem_cur, gsem_nxt, wsem_cur, wsem_nxt):
      base = base_w + ci * CHUNK

      @pl.when(ci < n_chunks - 1)
      def _():
        pltpu.async_copy(idx4_hbm.at[blk_w + ci + 1], idx_nxt, isem_nxt)

      @pl.when(ci > 0)
      def _():
        pltpu.make_async_copy(
            rows_nxt, out_hbm.at[pl.ds(base, CHUNK)], wsem_nxt).wait()

      @pl.when(ci < n_chunks - 1)
      def _():
        pltpu.make_async_copy(
            idx4_hbm.at[blk_w + ci + 1], idx_nxt, isem_nxt).wait()
        pltpu.async_copy(word_hbm.at[idx_nxt.at[0]], rows_nxt, gsem_nxt)

      pltpu.make_async_copy(
          word_hbm.at[idx_cur.at[0]], rows_cur, gsem_cur).wait()
      compute(idx_cur, rows_cur)
      pltpu.async_copy(rows_cur, out_hbm.at[pl.ds(base, CHUNK)], wsem_cur)

    pltpu.sync_copy(idx4_hbm.at[blk_w], idx0_v)
    pltpu.async_copy(word_hbm.at[idx0_v.at[0]], rows0_v, gsem0)

    def loop_body(i, c):
      ci = i * 2
      phase(ci, idx0_v, idx1_v, rows0_v, rows1_v,
            isem1, gsem0, gsem1, wsem0, wsem1)
      phase(ci + 1, idx1_v, idx0_v, rows1_v, rows0_v,
            isem0, gsem1, gsem0, wsem1, wsem0)
      return c

    lax.fori_loop(0, n_chunks // 2, loop_body, 0)
    last_base = base_w + (n_chunks - 1) * CHUNK
    pltpu.make_async_copy(
        rows1_v, out_hbm.at[pl.ds(last_base, CHUNK)], wsem1).wait()

  return k


def kernel(input_ids, position_ids, segment_ids, age_ids, word_table,
           pos_table, seg_table, age_table, ln_gamma, ln_beta):
  b, s = input_ids.shape
  n_tokens = b * s
  n_blocks = n_tokens // CHUNK
  idx4 = jnp.stack([
      input_ids.reshape(n_blocks, CHUNK),
      position_ids.reshape(n_blocks, CHUNK),
      segment_ids.reshape(n_blocks, CHUNK),
      age_ids.reshape(n_blocks, CHUNK),
  ], axis=1).astype(jnp.int32)
  small = jnp.concatenate(
      [pos_table, seg_table, age_table], axis=0).reshape(-1)
  gb = jnp.concatenate([ln_gamma, ln_beta], axis=0)
  pos_rows = pos_table.shape[0]
  seg_rows = seg_table.shape[0]
  small_rows = pos_rows + seg_rows + age_table.shape[0]
  k = _build(n_tokens, small_rows, pos_rows, seg_rows)
  out = k(idx4, word_table, small, gb)
  return out.reshape(b, s, HIDDEN)
